# baseline (speedup 1.0000x reference)
import math

import jax
import jax.numpy as jnp
from jax.experimental import pallas as pl
from jax.experimental.pallas import tpu as pltpu

_N = 20000
_NS = 20480
_K = 1000
_P = 1024
_IOU_THR = 0.7
_CANVAS_H = 800.0
_CANVAS_W = 1333.0
_BBOX_CLIP = math.log(1000.0 / 16.0)
_LVL_OFF = _CANVAS_W + _CANVAS_H
_HI_BITS = 0x3F800000

_HIGH = jax.lax.Precision.HIGHEST


def _dot(a, b, dims, precision=None):
    return jax.lax.dot_general(
        a, b, (dims, ((), ())), preferred_element_type=jnp.float32,
        precision=precision)


def _nms_kernel(keys_ref, keys8_ref, data_ref, out_ref, m_ref):
    i0 = jax.lax.broadcasted_iota(jnp.int32, (_P, _P), 0)
    i1 = jax.lax.broadcasted_iota(jnp.int32, (_P, _P), 1)
    ident = (i0 == i1).astype(jnp.float32)
    ustrict = (i0 < i1).astype(jnp.float32)

    keys8 = jax.lax.bitcast_convert_type(keys8_ref[:], jnp.int32)

    def bis_body(_, lohi):
        lo, hi = lohi
        mid = lo + (hi - lo) // 2
        cnt = jnp.sum((keys8 > mid).astype(jnp.int32))
        big = cnt >= _K
        return jnp.where(big, mid, lo), jnp.where(big, hi, mid)

    lo0 = jnp.int32(-1)
    hi0 = jnp.int32(_HI_BITS)
    _, thr = jax.lax.fori_loop(0, 31, bis_body, (lo0, hi0))

    keys = jax.lax.bitcast_convert_type(keys_ref[:], jnp.int32)
    m_gt = (keys > thr).astype(jnp.float32)
    m_eq = (keys == thr).astype(jnp.float32)
    need = jnp.float32(_K) - jnp.sum(m_gt)

    nchunk = _NS // _P
    eq_off = jnp.float32(0.0)
    sel_off = jnp.float32(0.0)
    sel_chunks = []
    dest_chunks = []
    for c in range(nchunk):
        sl = slice(c * _P, (c + 1) * _P)
        eqc = m_eq[:, sl]
        eq_ex = _dot(eqc, ustrict, ((1,), (0,))) + eq_off
        eq_off = eq_off + jnp.sum(eqc)
        selc = m_gt[:, sl] + eqc * (eq_ex < need).astype(jnp.float32)
        sel_ex = _dot(selc, ustrict, ((1,), (0,))) + sel_off
        sel_off = sel_off + jnp.sum(selc)
        sel_chunks.append(selc)
        dest_chunks.append(sel_ex)

    iota_col = i0[:, 0:1].astype(jnp.float32)
    acc = jnp.zeros((_P, 16), jnp.float32)
    for c in range(nchunk):
        pt = ((iota_col == dest_chunks[c]) * sel_chunks[c])
        acc = acc + _dot(pt, data_ref[c * _P:(c + 1) * _P, :],
                         ((1,), (0,)), _HIGH)

    kcol = acc[:, 8:9]
    krow = _dot(kcol, ident, ((0,), (0,)), _HIGH)
    before = ((krow > kcol) |
              ((krow == kcol) & (i1 < i0))).astype(jnp.float32)
    rank = _dot(before, jnp.ones((_P, 1), jnp.float32), ((1,), (0,)))
    p2 = (rank == i1.astype(jnp.float32)).astype(jnp.float32)
    sorted_col = _dot(p2, acc, ((0,), (0,)), _HIGH)
    rows = _dot(sorted_col, ident, ((0,), (0,)), _HIGH)

    dx, dy = rows[0:1], rows[1:2]
    dw, dh = rows[2:3], rows[3:4]
    px1, py1 = rows[4:5], rows[5:6]
    px2, py2 = rows[6:7], rows[7:8]
    sc, lv = rows[8:9], rows[9:10]

    ws = px2 - px1
    hs = py2 - py1
    cx = px1 + 0.5 * ws
    cy = py1 + 0.5 * hs
    dw = jnp.minimum(dw, _BBOX_CLIP)
    dh = jnp.minimum(dh, _BBOX_CLIP)
    pcx = dx * ws + cx
    pcy = dy * hs + cy
    pw = jnp.exp(dw) * ws
    ph = jnp.exp(dh) * hs
    x1 = jnp.clip(pcx - 0.5 * pw, 0.0, _CANVAS_W)
    y1 = jnp.clip(pcy - 0.5 * ph, 0.0, _CANVAS_H)
    x2 = jnp.clip(pcx + 0.5 * pw, 0.0, _CANVAS_W)
    y2 = jnp.clip(pcy + 0.5 * ph, 0.0, _CANVAS_H)

    small = ((x2 - x1) < 1e-2) | ((y2 - y1) < 1e-2)
    scores = jnp.where(small, -1.0, sc)

    off = lv * _LVL_OFF
    ox1, oy1 = x1 + off, y1 + off
    ox2, oy2 = x2 + off, y2 + off
    area = (ox2 - ox1) * (oy2 - oy1)

    cat5 = jnp.concatenate([ox1, oy1, ox2, oy2, area], axis=0)
    cols = _dot(ident, cat5, ((1,), (1,)), _HIGH)

    R = 256
    for c in range(_P // R):
        a_x1 = cols[c * R:(c + 1) * R, 0:1]
        a_y1 = cols[c * R:(c + 1) * R, 1:2]
        a_x2 = cols[c * R:(c + 1) * R, 2:3]
        a_y2 = cols[c * R:(c + 1) * R, 3:4]
        a_area = cols[c * R:(c + 1) * R, 4:5]
        ltx = jnp.maximum(a_x1, ox1)
        lty = jnp.maximum(a_y1, oy1)
        rbx = jnp.minimum(a_x2, ox2)
        rby = jnp.minimum(a_y2, oy2)
        wv = jnp.maximum(rbx - ltx, 0.0)
        hv = jnp.maximum(rby - lty, 0.0)
        inter = wv * hv
        union = a_area + area - inter
        iou = inter / jnp.maximum(union, 1e-9)
        ir = jax.lax.broadcasted_iota(jnp.int32, (R, _P), 0) + c * R
        jr = jax.lax.broadcasted_iota(jnp.int32, (R, _P), 1)
        mblk = ((iou > _IOU_THR) & (jr > ir)).astype(jnp.float32)
        m_ref[c * R:(c + 1) * R] = mblk

    sup0 = jnp.where(small, 1.0, 0.0)
    init_keep = 1.0 - sup0

    def fp_cond(carry):
        return carry[1]

    def fp_body(carry):
        k, _ = carry
        hit = _dot(k, m_ref[:], ((1,), (0,)))
        newk = init_keep * jnp.where(hit > 0.0, 0.0, 1.0)
        return newk, jnp.any(newk != k)

    keep, _ = jax.lax.while_loop(
        fp_cond, fp_body, (init_keep, jnp.bool_(True)))

    outrows = jnp.concatenate(
        [x1 * keep, y1 * keep, x2 * keep, y2 * keep, scores * keep,
         jnp.zeros((3, _P), jnp.float32)], axis=0)

    small_f = jnp.where(small, 1.0, 0.0)
    valid_f = 1.0 - small_f
    ex_valid = _dot(valid_f, ustrict, ((1,), (0,)))
    ex_small = _dot(small_f, ustrict, ((1,), (0,)))
    nvalid = jnp.sum(valid_f)
    dest = jnp.where(small, nvalid + ex_small, ex_valid)

    perm_t = (i0.astype(jnp.float32) == dest).astype(jnp.float32)
    out_ref[:] = _dot(outrows, perm_t, ((1,), (1,)), _HIGH)


def kernel(reg, priors, obj, levels):
    pad = _NS - _N
    keys = jnp.pad(obj, (0, pad), constant_values=-1.0)[None]
    keys8 = keys.reshape(8, _NS // 8)
    data = jnp.concatenate(
        [reg, priors, obj[:, None], levels.astype(jnp.float32)[:, None],
         jnp.zeros((_N, 6), jnp.float32)], axis=1)
    data = jnp.pad(data, ((0, pad), (0, 0)))

    out_t = pl.pallas_call(
        _nms_kernel,
        out_shape=jax.ShapeDtypeStruct((8, _P), jnp.float32),
        scratch_shapes=[pltpu.VMEM((_P, _P), jnp.float32)],
    )(keys, keys8, data)
    return out_t[:5, :_K].T


# speedup vs baseline: 2.2135x; 2.2135x over previous
import math

import jax
import jax.numpy as jnp
from jax.experimental import pallas as pl
from jax.experimental.pallas import tpu as pltpu

_K = 1000
_P = 1024
_IOU_THR = 0.7
_CANVAS_H = 800.0
_CANVAS_W = 1333.0
_BBOX_CLIP = math.log(1000.0 / 16.0)
_LVL_OFF = _CANVAS_W + _CANVAS_H


def _nms_kernel(in_ref, out_ref, m_ref):
    data = in_ref[:]
    dx, dy = data[0:1], data[1:2]
    dw, dh = data[2:3], data[3:4]
    px1, py1 = data[4:5], data[5:6]
    px2, py2 = data[6:7], data[7:8]
    sc, lv = data[8:9], data[9:10]

    ws = px2 - px1
    hs = py2 - py1
    cx = px1 + 0.5 * ws
    cy = py1 + 0.5 * hs
    dw = jnp.minimum(dw, _BBOX_CLIP)
    dh = jnp.minimum(dh, _BBOX_CLIP)
    pcx = dx * ws + cx
    pcy = dy * hs + cy
    pw = jnp.exp(dw) * ws
    ph = jnp.exp(dh) * hs
    x1 = jnp.clip(pcx - 0.5 * pw, 0.0, _CANVAS_W)
    y1 = jnp.clip(pcy - 0.5 * ph, 0.0, _CANVAS_H)
    x2 = jnp.clip(pcx + 0.5 * pw, 0.0, _CANVAS_W)
    y2 = jnp.clip(pcy + 0.5 * ph, 0.0, _CANVAS_H)

    small = ((x2 - x1) < 1e-2) | ((y2 - y1) < 1e-2)
    scores = jnp.where(small, -1.0, sc)

    off = lv * _LVL_OFF
    ox1, oy1 = x1 + off, y1 + off
    ox2, oy2 = x2 + off, y2 + off
    area = (ox2 - ox1) * (oy2 - oy1)

    i0 = jax.lax.broadcasted_iota(jnp.int32, (_P, _P), 0)
    i1 = jax.lax.broadcasted_iota(jnp.int32, (_P, _P), 1)
    ident = (i0 == i1).astype(jnp.float32)

    cat5 = jnp.concatenate([ox1, oy1, ox2, oy2, area], axis=0)
    cols = jax.lax.dot_general(
        ident, cat5, (((1,), (1,)), ((), ())),
        preferred_element_type=jnp.float32,
        precision=jax.lax.Precision.HIGHEST)

    R = 256
    for c in range(_P // R):
        a_x1 = cols[c * R:(c + 1) * R, 0:1]
        a_y1 = cols[c * R:(c + 1) * R, 1:2]
        a_x2 = cols[c * R:(c + 1) * R, 2:3]
        a_y2 = cols[c * R:(c + 1) * R, 3:4]
        a_area = cols[c * R:(c + 1) * R, 4:5]
        ltx = jnp.maximum(a_x1, ox1)
        lty = jnp.maximum(a_y1, oy1)
        rbx = jnp.minimum(a_x2, ox2)
        rby = jnp.minimum(a_y2, oy2)
        wv = jnp.maximum(rbx - ltx, 0.0)
        hv = jnp.maximum(rby - lty, 0.0)
        inter = wv * hv
        union = a_area + area - inter
        iou = inter / jnp.maximum(union, 1e-9)
        ir = jax.lax.broadcasted_iota(jnp.int32, (R, _P), 0) + c * R
        jr = jax.lax.broadcasted_iota(jnp.int32, (R, _P), 1)
        mblk = ((iou > _IOU_THR) & (jr > ir)).astype(jnp.float32)
        m_ref[c * R:(c + 1) * R] = mblk

    sup0 = jnp.where(small, 1.0, 0.0)
    init_keep = 1.0 - sup0

    def fp_cond(carry):
        return carry[1]

    def fp_body(carry):
        k, _ = carry
        hit = jax.lax.dot_general(
            k, m_ref[:], (((1,), (0,)), ((), ())),
            preferred_element_type=jnp.float32)
        newk = init_keep * jnp.where(hit > 0.0, 0.0, 1.0)
        return newk, jnp.any(newk != k)

    keep, _ = jax.lax.while_loop(
        fp_cond, fp_body, (init_keep, jnp.bool_(True)))

    rows = jnp.concatenate(
        [x1 * keep, y1 * keep, x2 * keep, y2 * keep, scores * keep,
         jnp.zeros((3, _P), jnp.float32)], axis=0)

    small_f = jnp.where(small, 1.0, 0.0)
    valid_f = 1.0 - small_f
    ustrict = (i0 < i1).astype(jnp.float32)
    ex_valid = jax.lax.dot_general(
        valid_f, ustrict, (((1,), (0,)), ((), ())),
        preferred_element_type=jnp.float32,
        precision=jax.lax.Precision.HIGHEST)
    ex_small = jax.lax.dot_general(
        small_f, ustrict, (((1,), (0,)), ((), ())),
        preferred_element_type=jnp.float32,
        precision=jax.lax.Precision.HIGHEST)
    nvalid = jnp.sum(valid_f)
    dest = jnp.where(small, nvalid + ex_small, ex_valid)

    jrow = jax.lax.broadcasted_iota(jnp.int32, (_P, _P), 0).astype(jnp.float32)
    perm_t = (jrow == dest).astype(jnp.float32)
    out_ref[:] = jax.lax.dot_general(
        rows, perm_t, (((1,), (1,)), ((), ())),
        preferred_element_type=jnp.float32,
        precision=jax.lax.Precision.HIGHEST)


def kernel(reg, priors, obj, levels):
    scores0, idx = jax.lax.top_k(obj, _K)
    reg_k = jnp.take(reg, idx, axis=0)
    pri_k = jnp.take(priors, idx, axis=0)
    lv_k = jnp.take(levels, idx, axis=0).astype(jnp.float32)

    pad = _P - _K
    reg_t = jnp.pad(reg_k, ((0, pad), (0, 0))).T
    pri_t = jnp.pad(pri_k, ((0, pad), (0, 0))).T
    sc_p = jnp.pad(scores0, (0, pad))[None]
    lv_p = jnp.pad(lv_k, (0, pad))[None]
    packed = jnp.concatenate(
        [reg_t, pri_t, sc_p, lv_p, jnp.zeros((6, _P), jnp.float32)], axis=0)

    out_t = pl.pallas_call(
        _nms_kernel,
        out_shape=jax.ShapeDtypeStruct((8, _P), jnp.float32),
        scratch_shapes=[pltpu.VMEM((_P, _P), jnp.float32)],
    )(packed)
    return out_t[:5, :_K].T
